# baseline (speedup 1.0000x reference)
import functools

import jax
import jax.numpy as jnp
from jax import lax
from jax.experimental import pallas as pl
from jax.experimental.pallas import tpu as pltpu
from jax.experimental.pallas import tpu_sc as plsc

_N = 10000
_F = 256
_HALF = 128
_NC = 2
_NS = 16
_CHUNK = 64
_NBUF = 4
_EPT = 10240
_EPAD = _NS * _EPT
_NCHUNK = _EPT // _CHUNK
_NPH = 4
_CPH = _NCHUNK // _NPH
_ACC = 10240
_RPT = _ACC // _NS
_WB = _RPT // _CHUNK
_TRASH = _N
_BR = 1000
_NB = _N // _BR
_EPS = 1e-5

def _make_mesh():
    return plsc.VectorSubcoreMesh(core_axis_name="c", subcore_axis_name="s")


def _sc_agg_body(hwl_hbm, src2_hbm, dst_hbm, zeros_hbm, out_hbm,
                 src_i, dst_i, b0, b1, b2, b3, acc_sh,
                 g0, g1, g2, g3, s0, s1, s2, s3):
    bufs = (b0, b1, b2, b3)
    gsems = (g0, g1, g2, g3)
    ssems = (s0, s1, s2, s3)
    c = lax.axis_index("c")
    s = lax.axis_index("s")
    pltpu.sync_copy(zeros_hbm, b0)
    row0 = s * _RPT
    for j in range(_WB):
        pltpu.sync_copy(b0, acc_sh.at[pl.ds(row0 + j * _CHUNK, _CHUNK)])
    plsc.subcore_barrier()

    for p in range(_NPH):
        pltpu.sync_copy(src2_hbm.at[c, s, p], src_i)
        pltpu.sync_copy(dst_hbm.at[s, p], dst_i)
        for b in range(_NBUF):
            pltpu.async_copy(hwl_hbm.at[src_i.at[b]], bufs[b], gsems[b])

        def body(i, carry):
            for b in range(_NBUF):
                cb = _NBUF * i + b
                pltpu.make_async_copy(hwl_hbm.at[src_i.at[cb]], bufs[b],
                                      gsems[b]).wait()
                pltpu.async_copy(bufs[b], acc_sh.at[dst_i.at[cb]], ssems[b],
                                 add=True)

                @pl.when(cb + _NBUF < _CPH)
                def _g(b=b, cb=cb):
                    pltpu.make_async_copy(bufs[b], acc_sh.at[dst_i.at[cb]],
                                          ssems[b]).wait()
                    pltpu.async_copy(hwl_hbm.at[src_i.at[cb + _NBUF]],
                                     bufs[b], gsems[b])

            return carry

        lax.fori_loop(0, _CPH // _NBUF, body, 0)
        for b in range(_NBUF):
            pltpu.make_async_copy(bufs[b], acc_sh.at[dst_i.at[b]],
                                  ssems[b]).wait()
    plsc.subcore_barrier()

    for j in range(_WB):
        r0 = row0 + j * _CHUNK
        b = bufs[j % _NBUF]
        if j >= _NBUF:
            pltpu.make_async_copy(b, out_hbm.at[c, pl.ds(r0, _CHUNK)],
                                  gsems[j % _NBUF]).wait()
        pltpu.sync_copy(acc_sh.at[pl.ds(r0, _CHUNK)], b)
        pltpu.async_copy(b, out_hbm.at[c, pl.ds(r0, _CHUNK)],
                         gsems[j % _NBUF])
    for j in range(max(0, _WB - _NBUF), _WB):
        pltpu.make_async_copy(bufs[j % _NBUF],
                              out_hbm.at[c, pl.ds(row0, _CHUNK)],
                              gsems[j % _NBUF]).wait()


def _sc_agg(hwl2, src2, dst_p, zeros_blk):
    f = pl.kernel(
        _sc_agg_body,
        out_type=jax.ShapeDtypeStruct((_NC, _ACC, _HALF), jnp.float32),
        mesh=_make_mesh(),
        scratch_types=(
            [pltpu.VMEM((_CPH, _CHUNK), jnp.int32),
             pltpu.VMEM((_CPH, _CHUNK), jnp.int32)]
            + [pltpu.VMEM((_CHUNK, _HALF), jnp.float32)] * _NBUF
            + [pltpu.VMEM_SHARED((_ACC, _HALF), jnp.float32)]
            + [pltpu.SemaphoreType.DMA] * (2 * _NBUF)
        ),
    )
    return f(hwl2, src2, dst_p, zeros_blk)


def _sc_cnt_body(dst_hbm, c16_hbm, out_hbm, dst_i, ones_v, buf_v, cnt_sh, sem):
    c = lax.axis_index("c")
    s = lax.axis_index("s")
    row0 = s * _RPT

    @pl.when(c == 0)
    def _zero():
        pltpu.sync_copy(c16_hbm.at[0], buf_v)
        pltpu.sync_copy(c16_hbm.at[1], ones_v)
        for j in range(_WB):
            pltpu.sync_copy(buf_v, cnt_sh.at[pl.ds(row0 + j * _CHUNK, _CHUNK)])
        pltpu.sync_copy(dst_hbm.at[s], dst_i)

    plsc.subcore_barrier()

    @pl.when(c == 0)
    def _accum():
        for p in range(_NPH):
            def fire(i, carry, p=p):
                pltpu.async_copy(ones_v, cnt_sh.at[dst_i.at[p, i]], sem,
                                 add=True)
                return carry
            lax.fori_loop(0, _CPH, fire, 0)

        for p in range(_NPH):
            def drain(i, carry, p=p):
                pltpu.make_async_copy(ones_v, cnt_sh.at[dst_i.at[p, i]],
                                      sem).wait()
                return carry
            lax.fori_loop(0, _CPH, drain, 0)

    plsc.subcore_barrier()

    @pl.when(c == 0)
    def _writeback():
        for j in range(_WB):
            r0 = row0 + j * _CHUNK
            pltpu.sync_copy(cnt_sh.at[pl.ds(r0, _CHUNK)], buf_v)
            pltpu.sync_copy(buf_v, out_hbm.at[pl.ds(r0, _CHUNK)])


def _sc_cnt(dst_p, c16):
    f = pl.kernel(
        _sc_cnt_body,
        out_type=jax.ShapeDtypeStruct((_ACC, 16), jnp.float32),
        mesh=_make_mesh(),
        scratch_types=[
            pltpu.VMEM((_NPH, _CPH, _CHUNK), jnp.int32),
            pltpu.VMEM((_CHUNK, 16), jnp.float32),
            pltpu.VMEM((_CHUNK, 16), jnp.float32),
            pltpu.VMEM_SHARED((_ACC, 16), jnp.float32),
            pltpu.SemaphoreType.DMA,
        ],
    )
    return f(dst_p, c16)


def _mm_split_body(x_ref, wl_ref, wr_ref, o1_ref, o2_ref):
    y = jnp.dot(x_ref[...], wl_ref[...], preferred_element_type=jnp.float32)
    o1_ref[0] = y[:, :_HALF]
    o1_ref[1] = y[:, _HALF:]
    o2_ref[...] = jnp.dot(x_ref[...], wr_ref[...],
                          preferred_element_type=jnp.float32)


def _mm_split(h, wl, wr):
    return pl.pallas_call(
        _mm_split_body,
        grid=(_NB,),
        in_specs=[
            pl.BlockSpec((_BR, _F), lambda i: (i, 0)),
            pl.BlockSpec((_F, _F), lambda i: (0, 0)),
            pl.BlockSpec((_F, _F), lambda i: (0, 0)),
        ],
        out_specs=[
            pl.BlockSpec((_NC, _BR, _HALF), lambda i: (0, i, 0)),
            pl.BlockSpec((_BR, _F), lambda i: (i, 0)),
        ],
        out_shape=[
            jax.ShapeDtypeStruct((_NC, _N, _HALF), jnp.float32),
            jax.ShapeDtypeStruct((_N, _F), jnp.float32),
        ],
    )(h, wl, wr)


def _post_body(agg_ref, cb_ref, hwr_ref, bl_ref, r_ref, st_ref):
    i = pl.program_id(0)
    cnt = jnp.maximum(cb_ref[...], 1.0)
    m0 = agg_ref[0] / cnt
    m1 = agg_ref[1] / cnt
    mean = jnp.concatenate([m0, m1], axis=1)
    r = jnp.maximum(mean + bl_ref[...] + hwr_ref[...], 0.0)
    r_ref[...] = r

    @pl.when(i == 0)
    def _init():
        st_ref[...] = jnp.zeros_like(st_ref)

    st_ref[...] += jnp.stack([jnp.sum(r, axis=0), jnp.sum(r * r, axis=0)])


def _post(agg, cnt_b, hwr, bl):
    return pl.pallas_call(
        _post_body,
        grid=(_NB,),
        in_specs=[
            pl.BlockSpec((_NC, _BR, _HALF), lambda i: (0, i, 0)),
            pl.BlockSpec((_BR, _HALF), lambda i: (i, 0)),
            pl.BlockSpec((_BR, _F), lambda i: (i, 0)),
            pl.BlockSpec((1, _F), lambda i: (0, 0)),
        ],
        out_specs=[
            pl.BlockSpec((_BR, _F), lambda i: (i, 0)),
            pl.BlockSpec((2, _F), lambda i: (0, 0)),
        ],
        out_shape=[
            jax.ShapeDtypeStruct((_N, _F), jnp.float32),
            jax.ShapeDtypeStruct((2, _F), jnp.float32),
        ],
    )(agg, cnt_b, hwr, bl)


def _norm_mm_body(r_ref, st_ref, g_ref, b_ref, wl_ref, wr_ref, o1_ref, o2_ref):
    mu = st_ref[0:1, :] * (1.0 / _N)
    var = st_ref[1:2, :] * (1.0 / _N) - mu * mu
    rstd = lax.rsqrt(var + _EPS)
    hn = (r_ref[...] - mu) * (rstd * g_ref[...]) + b_ref[...]
    y = jnp.dot(hn, wl_ref[...], preferred_element_type=jnp.float32)
    o1_ref[0] = y[:, :_HALF]
    o1_ref[1] = y[:, _HALF:]
    o2_ref[...] = jnp.dot(hn, wr_ref[...], preferred_element_type=jnp.float32)


def _norm_mm(r, st, g, b, wl, wr):
    return pl.pallas_call(
        _norm_mm_body,
        grid=(_NB,),
        in_specs=[
            pl.BlockSpec((_BR, _F), lambda i: (i, 0)),
            pl.BlockSpec((2, _F), lambda i: (0, 0)),
            pl.BlockSpec((1, _F), lambda i: (0, 0)),
            pl.BlockSpec((1, _F), lambda i: (0, 0)),
            pl.BlockSpec((_F, _F), lambda i: (0, 0)),
            pl.BlockSpec((_F, _F), lambda i: (0, 0)),
        ],
        out_specs=[
            pl.BlockSpec((_NC, _BR, _HALF), lambda i: (0, i, 0)),
            pl.BlockSpec((_BR, _F), lambda i: (i, 0)),
        ],
        out_shape=[
            jax.ShapeDtypeStruct((_NC, _N, _HALF), jnp.float32),
            jax.ShapeDtypeStruct((_N, _F), jnp.float32),
        ],
    )(r, st, g, b, wl, wr)


def _bn_body(r_ref, st_ref, g_ref, b_ref, o_ref):
    mu = st_ref[0:1, :] * (1.0 / _N)
    var = st_ref[1:2, :] * (1.0 / _N) - mu * mu
    rstd = lax.rsqrt(var + _EPS)
    o_ref[...] = (r_ref[...] - mu) * (rstd * g_ref[...]) + b_ref[...]


def _bn(r, st, g, b):
    return pl.pallas_call(
        _bn_body,
        grid=(_NB,),
        in_specs=[
            pl.BlockSpec((_BR, _F), lambda i: (i, 0)),
            pl.BlockSpec((2, _F), lambda i: (0, 0)),
            pl.BlockSpec((1, _F), lambda i: (0, 0)),
            pl.BlockSpec((1, _F), lambda i: (0, 0)),
        ],
        out_specs=pl.BlockSpec((_BR, _F), lambda i: (i, 0)),
        out_shape=jax.ShapeDtypeStruct((_N, _F), jnp.float32),
    )(r, st, g, b)


def kernel(x, edge_index, edge_attr, Wl1, bl1, Wr1, g1, b1,
           Wl2, bl2, Wr2, g2, b2, Wl3, bl3, Wr3, g3, b3):
    ei = edge_index.astype(jnp.int32)
    e = ei.shape[1]
    pad = _EPAD - e
    src_p = jnp.concatenate([ei[0], jnp.zeros((pad,), jnp.int32)])
    dst_p = jnp.concatenate([ei[1], jnp.full((pad,), _TRASH, jnp.int32)])
    src2 = jnp.stack([src_p, src_p + _N]).reshape(_NC, _NS, _NPH, _CPH, _CHUNK)
    dst_p = dst_p.reshape(_NS, _NPH, _CPH, _CHUNK)

    zeros_blk = jnp.zeros((_CHUNK, _HALF), jnp.float32)
    c16 = jnp.stack([jnp.zeros((_CHUNK, 16), jnp.float32),
                     jnp.ones((_CHUNK, 16), jnp.float32)])

    cnt_out = _sc_cnt(dst_p, c16)
    cnt_b = jnp.broadcast_to(cnt_out[:_N, 0:1], (_N, _HALF))

    bls = (bl1, bl2, bl3)
    gs = (g1, g2, g3)
    bs = (b1, b2, b3)
    wnext = ((Wl2, Wr2), (Wl3, Wr3))

    hwl, hwr = _mm_split(x, Wl1, Wr1)
    r = st = None
    for layer in range(3):
        agg = _sc_agg(hwl.reshape(_NC * _N, _HALF), src2, dst_p, zeros_blk)
        r, st = _post(agg, cnt_b, hwr, bls[layer].reshape(1, _F))
        if layer < 2:
            hwl, hwr = _norm_mm(r, st, gs[layer].reshape(1, _F),
                                bs[layer].reshape(1, _F), *wnext[layer])
    return _bn(r, st, g3.reshape(1, _F), b3.reshape(1, _F))


# speedup vs baseline: 1.0654x; 1.0654x over previous
import functools

import jax
import jax.numpy as jnp
from jax import lax
from jax.experimental import pallas as pl
from jax.experimental.pallas import tpu as pltpu
from jax.experimental.pallas import tpu_sc as plsc

_N = 10000
_F = 256
_HALF = 128
_NC = 2
_NS = 16
_CHUNK = 64
_NBUF = 4
_EPT = 10240
_EPAD = _NS * _EPT
_NCHUNK = _EPT // _CHUNK
_NPH = 4
_CPH = _NCHUNK // _NPH
_ACC = 10240
_RPT = _ACC // _NS
_WB = _RPT // _CHUNK
_TRASH = _N
_BR = 400
_NB = _N // _BR
_EPS = 1e-5

def _make_mesh():
    return plsc.VectorSubcoreMesh(core_axis_name="c", subcore_axis_name="s")


def _sc_agg_body(hwl_hbm, src2_hbm, dst_hbm, zeros_hbm, out_hbm,
                 src_i, dst_i, b0, b1, b2, b3, acc_sh,
                 g0, g1, g2, g3, s0, s1, s2, s3):
    bufs = (b0, b1, b2, b3)
    gsems = (g0, g1, g2, g3)
    ssems = (s0, s1, s2, s3)
    c = lax.axis_index("c")
    s = lax.axis_index("s")
    pltpu.sync_copy(zeros_hbm, b0)
    row0 = s * _RPT
    for j in range(_WB):
        pltpu.sync_copy(b0, acc_sh.at[pl.ds(row0 + j * _CHUNK, _CHUNK)])
    plsc.subcore_barrier()

    for p in range(_NPH):
        pltpu.sync_copy(src2_hbm.at[c, s, p], src_i)
        pltpu.sync_copy(dst_hbm.at[s, p], dst_i)
        for b in range(_NBUF):
            pltpu.async_copy(hwl_hbm.at[src_i.at[b]], bufs[b], gsems[b])

        def body(i, carry):
            for b in range(_NBUF):
                cb = _NBUF * i + b
                pltpu.make_async_copy(hwl_hbm.at[src_i.at[cb]], bufs[b],
                                      gsems[b]).wait()
                pltpu.async_copy(bufs[b], acc_sh.at[dst_i.at[cb]], ssems[b],
                                 add=True)

                @pl.when(cb + _NBUF < _CPH)
                def _g(b=b, cb=cb):
                    pltpu.make_async_copy(bufs[b], acc_sh.at[dst_i.at[cb]],
                                          ssems[b]).wait()
                    pltpu.async_copy(hwl_hbm.at[src_i.at[cb + _NBUF]],
                                     bufs[b], gsems[b])

            return carry

        lax.fori_loop(0, _CPH // _NBUF, body, 0)
        for b in range(_NBUF):
            pltpu.make_async_copy(bufs[b], acc_sh.at[dst_i.at[b]],
                                  ssems[b]).wait()
    plsc.subcore_barrier()

    for j in range(_WB):
        r0 = row0 + j * _CHUNK
        b = bufs[j % _NBUF]
        if j >= _NBUF:
            pltpu.make_async_copy(b, out_hbm.at[c, pl.ds(r0, _CHUNK)],
                                  gsems[j % _NBUF]).wait()
        pltpu.sync_copy(acc_sh.at[pl.ds(r0, _CHUNK)], b)
        pltpu.async_copy(b, out_hbm.at[c, pl.ds(r0, _CHUNK)],
                         gsems[j % _NBUF])
    for j in range(max(0, _WB - _NBUF), _WB):
        pltpu.make_async_copy(bufs[j % _NBUF],
                              out_hbm.at[c, pl.ds(row0, _CHUNK)],
                              gsems[j % _NBUF]).wait()


def _sc_agg(hwl2, src2, dst_p, zeros_blk):
    f = pl.kernel(
        _sc_agg_body,
        out_type=jax.ShapeDtypeStruct((_NC, _ACC, _HALF), jnp.float32),
        mesh=_make_mesh(),
        scratch_types=(
            [pltpu.VMEM((_CPH, _CHUNK), jnp.int32),
             pltpu.VMEM((_CPH, _CHUNK), jnp.int32)]
            + [pltpu.VMEM((_CHUNK, _HALF), jnp.float32)] * _NBUF
            + [pltpu.VMEM_SHARED((_ACC, _HALF), jnp.float32)]
            + [pltpu.SemaphoreType.DMA] * (2 * _NBUF)
        ),
    )
    return f(hwl2, src2, dst_p, zeros_blk)


def _sc_cnt_body(dst_hbm, c16_hbm, out_hbm, dst_i, ones_v, buf_v, cnt_sh, sem):
    c = lax.axis_index("c")
    s = lax.axis_index("s")
    row0 = s * _RPT

    @pl.when(c == 0)
    def _zero():
        pltpu.sync_copy(c16_hbm.at[0], buf_v)
        pltpu.sync_copy(c16_hbm.at[1], ones_v)
        for j in range(_WB):
            pltpu.sync_copy(buf_v, cnt_sh.at[pl.ds(row0 + j * _CHUNK, _CHUNK)])
        pltpu.sync_copy(dst_hbm.at[s], dst_i)

    plsc.subcore_barrier()

    @pl.when(c == 0)
    def _accum():
        for p in range(_NPH):
            def fire(i, carry, p=p):
                pltpu.async_copy(ones_v, cnt_sh.at[dst_i.at[p, i]], sem,
                                 add=True)
                return carry
            lax.fori_loop(0, _CPH, fire, 0)

        for p in range(_NPH):
            def drain(i, carry, p=p):
                pltpu.make_async_copy(ones_v, cnt_sh.at[dst_i.at[p, i]],
                                      sem).wait()
                return carry
            lax.fori_loop(0, _CPH, drain, 0)

    plsc.subcore_barrier()

    @pl.when(c == 0)
    def _writeback():
        for j in range(_WB):
            r0 = row0 + j * _CHUNK
            pltpu.sync_copy(cnt_sh.at[pl.ds(r0, _CHUNK)], buf_v)
            pltpu.sync_copy(buf_v, out_hbm.at[pl.ds(r0, _CHUNK)])


def _sc_cnt(dst_p, c16):
    f = pl.kernel(
        _sc_cnt_body,
        out_type=jax.ShapeDtypeStruct((_ACC, 16), jnp.float32),
        mesh=_make_mesh(),
        scratch_types=[
            pltpu.VMEM((_NPH, _CPH, _CHUNK), jnp.int32),
            pltpu.VMEM((_CHUNK, 16), jnp.float32),
            pltpu.VMEM((_CHUNK, 16), jnp.float32),
            pltpu.VMEM_SHARED((_ACC, 16), jnp.float32),
            pltpu.SemaphoreType.DMA,
        ],
    )
    return f(dst_p, c16)


def _mm_l_body(x_ref, wl_ref, o1_ref):
    y = jnp.dot(x_ref[...], wl_ref[...], preferred_element_type=jnp.float32)
    o1_ref[0] = y[:, :_HALF]
    o1_ref[1] = y[:, _HALF:]


def _mm_l(h, wl):
    return pl.pallas_call(
        _mm_l_body,
        grid=(_NB,),
        in_specs=[
            pl.BlockSpec((_BR, _F), lambda i: (i, 0)),
            pl.BlockSpec((_F, _F), lambda i: (0, 0)),
        ],
        out_specs=pl.BlockSpec((_NC, _BR, _HALF), lambda i: (0, i, 0)),
        out_shape=jax.ShapeDtypeStruct((_NC, _N, _HALF), jnp.float32),
    )(h, wl)


def _mm_r_body(x_ref, wr_ref, o2_ref):
    o2_ref[...] = jnp.dot(x_ref[...], wr_ref[...],
                          preferred_element_type=jnp.float32)


def _mm_r(h, wr):
    return pl.pallas_call(
        _mm_r_body,
        grid=(_NB,),
        in_specs=[
            pl.BlockSpec((_BR, _F), lambda i: (i, 0)),
            pl.BlockSpec((_F, _F), lambda i: (0, 0)),
        ],
        out_specs=pl.BlockSpec((_BR, _F), lambda i: (i, 0)),
        out_shape=jax.ShapeDtypeStruct((_N, _F), jnp.float32),
    )(h, wr)


def _post_body(agg_ref, cb_ref, hwr_ref, bl_ref, r_ref, st_ref):
    i = pl.program_id(0)
    cnt = jnp.maximum(cb_ref[...], 1.0)
    m0 = agg_ref[0] / cnt
    m1 = agg_ref[1] / cnt
    mean = jnp.concatenate([m0, m1], axis=1)
    r = jnp.maximum(mean + bl_ref[...] + hwr_ref[...], 0.0)
    r_ref[...] = r

    @pl.when(i == 0)
    def _init():
        st_ref[...] = jnp.zeros_like(st_ref)

    st_ref[...] += jnp.stack([jnp.sum(r, axis=0), jnp.sum(r * r, axis=0)])


def _post(agg, cnt_b, hwr, bl):
    return pl.pallas_call(
        _post_body,
        grid=(_NB,),
        in_specs=[
            pl.BlockSpec((_NC, _BR, _HALF), lambda i: (0, i, 0)),
            pl.BlockSpec((_BR, _HALF), lambda i: (i, 0)),
            pl.BlockSpec((_BR, _F), lambda i: (i, 0)),
            pl.BlockSpec((1, _F), lambda i: (0, 0)),
        ],
        out_specs=[
            pl.BlockSpec((_BR, _F), lambda i: (i, 0)),
            pl.BlockSpec((2, _F), lambda i: (0, 0)),
        ],
        out_shape=[
            jax.ShapeDtypeStruct((_N, _F), jnp.float32),
            jax.ShapeDtypeStruct((2, _F), jnp.float32),
        ],
    )(agg, cnt_b, hwr, bl)


def _norm_mm_l_body(r_ref, st_ref, g_ref, b_ref, wl_ref, o1_ref):
    mu = st_ref[0:1, :] * (1.0 / _N)
    var = st_ref[1:2, :] * (1.0 / _N) - mu * mu
    rstd = lax.rsqrt(var + _EPS)
    hn = (r_ref[...] - mu) * (rstd * g_ref[...]) + b_ref[...]
    y = jnp.dot(hn, wl_ref[...], preferred_element_type=jnp.float32)
    o1_ref[0] = y[:, :_HALF]
    o1_ref[1] = y[:, _HALF:]


def _norm_mm_l(r, st, g, b, wl):
    return pl.pallas_call(
        _norm_mm_l_body,
        grid=(_NB,),
        in_specs=[
            pl.BlockSpec((_BR, _F), lambda i: (i, 0)),
            pl.BlockSpec((2, _F), lambda i: (0, 0)),
            pl.BlockSpec((1, _F), lambda i: (0, 0)),
            pl.BlockSpec((1, _F), lambda i: (0, 0)),
            pl.BlockSpec((_F, _F), lambda i: (0, 0)),
        ],
        out_specs=pl.BlockSpec((_NC, _BR, _HALF), lambda i: (0, i, 0)),
        out_shape=jax.ShapeDtypeStruct((_NC, _N, _HALF), jnp.float32),
    )(r, st, g, b, wl)


def _norm_mm_r_body(r_ref, st_ref, g_ref, b_ref, wr_ref, o2_ref):
    mu = st_ref[0:1, :] * (1.0 / _N)
    var = st_ref[1:2, :] * (1.0 / _N) - mu * mu
    rstd = lax.rsqrt(var + _EPS)
    hn = (r_ref[...] - mu) * (rstd * g_ref[...]) + b_ref[...]
    o2_ref[...] = jnp.dot(hn, wr_ref[...], preferred_element_type=jnp.float32)


def _norm_mm_r(r, st, g, b, wr):
    return pl.pallas_call(
        _norm_mm_r_body,
        grid=(_NB,),
        in_specs=[
            pl.BlockSpec((_BR, _F), lambda i: (i, 0)),
            pl.BlockSpec((2, _F), lambda i: (0, 0)),
            pl.BlockSpec((1, _F), lambda i: (0, 0)),
            pl.BlockSpec((1, _F), lambda i: (0, 0)),
            pl.BlockSpec((_F, _F), lambda i: (0, 0)),
        ],
        out_specs=pl.BlockSpec((_BR, _F), lambda i: (i, 0)),
        out_shape=jax.ShapeDtypeStruct((_N, _F), jnp.float32),
    )(r, st, g, b, wr)


def _bn_body(r_ref, st_ref, g_ref, b_ref, o_ref):
    mu = st_ref[0:1, :] * (1.0 / _N)
    var = st_ref[1:2, :] * (1.0 / _N) - mu * mu
    rstd = lax.rsqrt(var + _EPS)
    o_ref[...] = (r_ref[...] - mu) * (rstd * g_ref[...]) + b_ref[...]


def _bn(r, st, g, b):
    return pl.pallas_call(
        _bn_body,
        grid=(_NB,),
        in_specs=[
            pl.BlockSpec((_BR, _F), lambda i: (i, 0)),
            pl.BlockSpec((2, _F), lambda i: (0, 0)),
            pl.BlockSpec((1, _F), lambda i: (0, 0)),
            pl.BlockSpec((1, _F), lambda i: (0, 0)),
        ],
        out_specs=pl.BlockSpec((_BR, _F), lambda i: (i, 0)),
        out_shape=jax.ShapeDtypeStruct((_N, _F), jnp.float32),
    )(r, st, g, b)


def kernel(x, edge_index, edge_attr, Wl1, bl1, Wr1, g1, b1,
           Wl2, bl2, Wr2, g2, b2, Wl3, bl3, Wr3, g3, b3):
    ei = edge_index.astype(jnp.int32)
    e = ei.shape[1]
    pad = _EPAD - e
    src_p = jnp.concatenate([ei[0], jnp.zeros((pad,), jnp.int32)])
    dst_p = jnp.concatenate([ei[1], jnp.full((pad,), _TRASH, jnp.int32)])
    src2 = jnp.stack([src_p, src_p + _N]).reshape(_NC, _NS, _NPH, _CPH, _CHUNK)
    dst_p = dst_p.reshape(_NS, _NPH, _CPH, _CHUNK)

    zeros_blk = jnp.zeros((_CHUNK, _HALF), jnp.float32)
    c16 = jnp.stack([jnp.zeros((_CHUNK, 16), jnp.float32),
                     jnp.ones((_CHUNK, 16), jnp.float32)])

    cnt_out = _sc_cnt(dst_p, c16)
    cnt_b = jnp.broadcast_to(cnt_out[:_N, 0:1], (_N, _HALF))

    bls = (bl1, bl2, bl3)
    gs = (g1, g2, g3)
    bs = (b1, b2, b3)
    wnext = ((Wl2, Wr2), (Wl3, Wr3))

    hwl = _mm_l(x, Wl1)
    r = st = None
    for layer in range(3):
        agg = _sc_agg(hwl.reshape(_NC * _N, _HALF), src2, dst_p, zeros_blk)
        if layer == 0:
            hwr = _mm_r(x, Wr1)
        else:
            hwr = _norm_mm_r(r, st, gs[layer - 1].reshape(1, _F),
                             bs[layer - 1].reshape(1, _F), wnext[layer - 1][1])
        r, st = _post(agg, cnt_b, hwr, bls[layer].reshape(1, _F))
        if layer < 2:
            hwl = _norm_mm_l(r, st, gs[layer].reshape(1, _F),
                             bs[layer].reshape(1, _F), wnext[layer][0])
    return _bn(r, st, g3.reshape(1, _F), b3.reshape(1, _F))
